# baseline (speedup 1.0000x reference)
import functools

import jax
import jax.numpy as jnp
from jax import lax
from jax.experimental import pallas as pl
from jax.experimental.pallas import tpu as pltpu
from jax.experimental.pallas import tpu_sc as plsc

B = 16
L = 2048
K = 8
FA = 128
M = 12
FN = 64
DV = 256
NC = 2
NS = 16
NW = NC * NS
PARTS = NW // K
ROWS = L // PARTS
IDX_CHUNK = ROWS * M


def _sc_body(atom_hbm, nbr_hbm, idx_hbm, vft_hbm, w_hbm, tgt_hbm,
             out_atom, out_nbr, out_idx, out_tgt,
             vft_v, w_v, idx_v, tgt_v, outtgt_v, sem):
    wid = lax.axis_index("s") * NC + lax.axis_index("c")
    r = wid // PARTS
    part = wid % PARTS

    pltpu.sync_copy(vft_hbm, vft_v)
    pltpu.sync_copy(w_hbm, w_v)

    def fma(c, acc):
        wchunk = w_v[pl.ds(c * B, B)]
        for lane in range(B):
            acc = acc + wchunk[lane] * vft_v[pl.ds((c * B + lane) * B, B)]
        return acc

    scores = lax.fori_loop(0, DV // B, fma, jnp.zeros((B,), jnp.float32))

    s = [scores[j] for j in range(B)]
    rank = []
    for j in range(B):
        rj = jnp.int32(0)
        for i in range(B):
            if i == j:
                continue
            beats = (s[i] > s[j]) | ((s[i] == s[j]) & (i < j))
            rj = rj + jnp.where(beats, 1, 0).astype(jnp.int32)
        rank.append(rj)

    sel_r = jnp.int32(0)
    for j in range(B):
        sel_r = sel_r + jnp.where(rank[j] == r, j, 0).astype(jnp.int32)
    delta = (r - sel_r) * L

    src0 = sel_r * L + part * ROWS
    dst0 = r * L + part * ROWS

    atom_cp = pltpu.async_copy(
        atom_hbm.at[pl.ds(src0, ROWS)], out_atom.at[pl.ds(dst0, ROWS)], sem)
    nbr_cp = pltpu.async_copy(
        nbr_hbm.at[pl.ds(src0, ROWS)], out_nbr.at[pl.ds(dst0, ROWS)], sem)

    isrc = sel_r * (L * M) + part * IDX_CHUNK
    idst = r * (L * M) + part * IDX_CHUNK
    pltpu.sync_copy(idx_hbm.at[pl.ds(isrc, IDX_CHUNK)], idx_v)

    def add_delta(i, _):
        sl = pl.ds(i * B, B)
        idx_v[sl] = idx_v[sl] + delta
        return 0

    lax.fori_loop(0, IDX_CHUNK // B, add_delta, 0)
    pltpu.sync_copy(idx_v, out_idx.at[pl.ds(idst, IDX_CHUNK)])

    @pl.when(wid == 0)
    def _():
        pltpu.sync_copy(tgt_hbm, tgt_v)
        t = tgt_v[...]
        iota = lax.iota(jnp.int32, B)
        out_vec = jnp.zeros((B,), jnp.float32)
        for j in range(B):
            out_vec = jnp.where(iota == rank[j], t[j], out_vec)
        outtgt_v[...] = out_vec
        pltpu.sync_copy(outtgt_v.at[pl.ds(0, K)], out_tgt)

    atom_cp.wait()
    nbr_cp.wait()


@jax.jit
def _run(atom_fea, nbr2d, idx_flat, vft, w, tgt):
    kern = pl.kernel(
        _sc_body,
        out_type=[
            jax.ShapeDtypeStruct((K * L, FA), jnp.float32),
            jax.ShapeDtypeStruct((K * L, M * FN), jnp.float32),
            jax.ShapeDtypeStruct((K * L * M,), jnp.int32),
            jax.ShapeDtypeStruct((K,), jnp.float32),
        ],
        mesh=plsc.VectorSubcoreMesh(core_axis_name="c", subcore_axis_name="s"),
        scratch_types=[
            pltpu.VMEM((DV * B,), jnp.float32),
            pltpu.VMEM((DV,), jnp.float32),
            pltpu.VMEM((IDX_CHUNK,), jnp.int32),
            pltpu.VMEM((B,), jnp.float32),
            pltpu.VMEM((B,), jnp.float32),
            pltpu.SemaphoreType.DMA,
        ],
    )
    return kern(atom_fea, nbr2d, idx_flat, vft, w, tgt)


def kernel(atom_fea, nbr_fea, nbr_fea_idx, cu_seqlens, val_feat, target, W_val):
    total = atom_fea.shape[0]
    nbr2d = nbr_fea.reshape(total, M * FN)
    idx_flat = nbr_fea_idx.reshape(total * M)
    vft = val_feat.T.reshape(DV * B)
    w = W_val.reshape(DV)
    tgt = target.reshape(B)
    na, nn, ni, nt = _run(atom_fea, nbr2d, idx_flat, vft, w, tgt)
    return (na, nn.reshape(K * L, M, FN), ni.reshape(K * L, M),
            nt.reshape(K, 1))
